# baseline (speedup 1.0000x reference)
import jax
import jax.numpy as jnp
from jax.experimental import pallas as pl
from jax.experimental.pallas import tpu as pltpu

D = 4096

_DN = (((1,), (1,)), ((), ()))


def _dot(a, b):
    return jax.lax.dot_general(a, b, _DN, preferred_element_type=jnp.float32)


def _body(x_ref, t_ref, W1_ref, b1_ref, W2_ref, b2_ref,
          Wc1_hbm, bc1_ref, Wc2_hbm, bc2_ref, reg_ref,
          imp_ref, cons_ref, loss_ref, wc1_v, wc2_v, sems):
    cp_wc1 = pltpu.make_async_copy(Wc1_hbm, wc1_v, sems.at[0])
    cp_wc2 = pltpu.make_async_copy(Wc2_hbm, wc2_v, sems.at[1])
    cp_wc1.start()
    cp_wc2.start()

    x = x_ref[...]
    t = t_ref[...]

    h = _dot(x, W1_ref[:, :D]) + _dot(t, W1_ref[:, D:])
    h = jnp.maximum(h + b1_ref[...], 0.0)
    logit = jnp.sum(h * W2_ref[...]) + b2_ref[0]
    imp = jax.nn.sigmoid(logit)
    imp_ref[0] = imp
    gate = jnp.where(imp > 0.5, jnp.float32(1.0), jnp.float32(0.0))

    cp_wc1.wait()
    hc = jnp.maximum(_dot(x, wc1_v[...]) + bc1_ref[...], 0.0)

    cp_wc2.wait()
    cons = _dot(hc, wc2_v[...]) + bc2_ref[...]
    cons_ref[...] = cons * gate

    dist = jnp.sum((x - x) ** 2)
    loss_ref[0] = jnp.where(imp > 0.5, reg_ref[0] * (imp * dist),
                            jnp.float32(0.0))


def kernel(current_features, target, W1, b1, W2, b2, Wc1, bc1, Wc2, bc2,
           reg_controller):
    x = current_features.reshape(1, D)
    t = target.reshape(1, D)
    smem = pl.BlockSpec(memory_space=pltpu.SMEM)
    imp, cons, loss = pl.pallas_call(
        _body,
        out_shape=(
            jax.ShapeDtypeStruct((1,), jnp.float32),
            jax.ShapeDtypeStruct((1, D), jnp.float32),
            jax.ShapeDtypeStruct((1,), jnp.float32),
        ),
        in_specs=[pl.BlockSpec((1, D), lambda: (0, 0)),
                  pl.BlockSpec((1, D), lambda: (0, 0)),
                  pl.BlockSpec((128, 2 * D), lambda: (0, 0)),
                  pl.BlockSpec((1, 128), lambda: (0, 0)),
                  pl.BlockSpec((1, 128), lambda: (0, 0)),
                  smem,
                  pl.BlockSpec(memory_space=pl.ANY),
                  pl.BlockSpec((1, 256), lambda: (0, 0)),
                  pl.BlockSpec(memory_space=pl.ANY),
                  pl.BlockSpec((1, D), lambda: (0, 0)),
                  smem],
        out_specs=(smem,
                   pl.BlockSpec((1, D), lambda: (0, 0)),
                   smem),
        scratch_shapes=[pltpu.VMEM((256, D), jnp.float32),
                        pltpu.VMEM((D, 256), jnp.float32),
                        pltpu.SemaphoreType.DMA((2,))],
    )(x, t, W1, b1.reshape(1, 128), W2, b2,
      Wc1, bc1.reshape(1, 256), Wc2, bc2.reshape(1, D),
      reg_controller.reshape(1))
    return imp, cons.reshape(D), loss.reshape(())


# speedup vs baseline: 1.0626x; 1.0626x over previous
import jax
import jax.numpy as jnp
from jax.experimental import pallas as pl
from jax.experimental.pallas import tpu as pltpu

D = 4096

_DN = (((1,), (1,)), ((), ()))


def _dot(a, b):
    return jax.lax.dot_general(a, b, _DN, preferred_element_type=jnp.float32)


def _body(x_ref, t_ref, W1_ref, b1_ref, W2_ref, b2_ref,
          Wc1_ref, bc1_ref, Wc2_hbm, bc2_ref, reg_ref,
          imp_ref, cons_ref, loss_ref, wc2_v, sems):
    H = D // 2
    cp_a = pltpu.make_async_copy(Wc2_hbm.at[pl.ds(0, H), :],
                                 wc2_v.at[pl.ds(0, H), :], sems.at[0])
    cp_b = pltpu.make_async_copy(Wc2_hbm.at[pl.ds(H, H), :],
                                 wc2_v.at[pl.ds(H, H), :], sems.at[1])
    cp_a.start()
    cp_b.start()

    x = x_ref[...]
    t = t_ref[...]

    h = _dot(x, W1_ref[:, :D]) + _dot(t, W1_ref[:, D:])
    h = jnp.maximum(h + b1_ref[...], 0.0)
    logit = jnp.sum(h * W2_ref[...]) + b2_ref[0]
    imp = jax.nn.sigmoid(logit)
    imp_ref[0] = imp
    gate = jnp.where(imp > 0.5, jnp.float32(1.0), jnp.float32(0.0))

    hc = jnp.maximum(_dot(x, Wc1_ref[...]) + bc1_ref[...], 0.0)

    cp_a.wait()
    cons_a = _dot(hc, wc2_v[pl.ds(0, H), :]) + bc2_ref[:, :H]
    cons_ref[:, :H] = cons_a * gate
    cp_b.wait()
    cons_b = _dot(hc, wc2_v[pl.ds(H, H), :]) + bc2_ref[:, H:]
    cons_ref[:, H:] = cons_b * gate

    dist = jnp.sum((x - x) ** 2)
    loss_ref[0] = jnp.where(imp > 0.5, reg_ref[0] * (imp * dist),
                            jnp.float32(0.0))


def kernel(current_features, target, W1, b1, W2, b2, Wc1, bc1, Wc2, bc2,
           reg_controller):
    x = current_features.reshape(1, D)
    t = target.reshape(1, D)
    smem = pl.BlockSpec(memory_space=pltpu.SMEM)
    imp, cons, loss = pl.pallas_call(
        _body,
        out_shape=(
            jax.ShapeDtypeStruct((1,), jnp.float32),
            jax.ShapeDtypeStruct((1, D), jnp.float32),
            jax.ShapeDtypeStruct((1,), jnp.float32),
        ),
        in_specs=[pl.BlockSpec((1, D), lambda: (0, 0)),
                  pl.BlockSpec((1, D), lambda: (0, 0)),
                  pl.BlockSpec((128, 2 * D), lambda: (0, 0)),
                  pl.BlockSpec((1, 128), lambda: (0, 0)),
                  pl.BlockSpec((1, 128), lambda: (0, 0)),
                  smem,
                  pl.BlockSpec((256, D), lambda: (0, 0)),
                  pl.BlockSpec((1, 256), lambda: (0, 0)),
                  pl.BlockSpec(memory_space=pl.ANY),
                  pl.BlockSpec((1, D), lambda: (0, 0)),
                  smem],
        out_specs=(smem,
                   pl.BlockSpec((1, D), lambda: (0, 0)),
                   smem),
        scratch_shapes=[pltpu.VMEM((D, 256), jnp.float32),
                        pltpu.SemaphoreType.DMA((2,))],
    )(x, t, W1, b1.reshape(1, 128), W2, b2,
      Wc1, bc1.reshape(1, 256), Wc2, bc2.reshape(1, D),
      reg_controller.reshape(1))
    return imp, cons.reshape(D), loss.reshape(())
